# Initial kernel scaffold; baseline (speedup 1.0000x reference)
import jax
import jax.numpy as jnp
from jax.experimental import pallas as pl


def kernel(z_gat, z_gin, gate_W, gate_b, e0_fc1_W, e0_fc1_b, e0_fc2_W, e0_fc2_b, e1_fc1_W, e1_fc1_b, e1_fc2_W, e1_fc2_b, e2_in_W, e2_in_b, e2_out_W, e2_out_b, e2_fc_W, e2_fc_b, e3_alpha_W, e3_alpha_b, e3_out_W, e3_out_b):
    raise NotImplementedError("write your pallas kernel here")


# speedup vs baseline: 2.2736x; 2.2736x over previous
import functools

import jax
import jax.numpy as jnp
from jax.experimental import pallas as pl
from jax.experimental.pallas import tpu as pltpu

D = 1024
B = 8192
NH = 4
HD = D // NH

TG = 2048
TE = 512


def _mm(a, w, b=None):
    out = jax.lax.dot_general(a, w, (((1,), (1,)), ((), ())),
                              preferred_element_type=jnp.float32)
    if b is not None:
        out = out + b
    return out


def _gate_kernel(zg_ref, zi_ref, gw_ref, gb_ref, p_ref, cnt_ref, psum_ref):
    i = pl.program_id(0)
    x = jnp.concatenate([zg_ref[...], zi_ref[...]], axis=1)
    logits = _mm(x, gw_ref[...], gb_ref[...])
    m = jnp.max(logits, axis=1, keepdims=True)
    e = jnp.exp(logits - m)
    probs = e / jnp.sum(e, axis=1, keepdims=True)
    pmax = jnp.max(probs, axis=1, keepdims=True)
    eqf = (probs == pmax).astype(jnp.float32)
    c0, c1, c2 = eqf[:, 0:1], eqf[:, 1:2], eqf[:, 2:3]
    prior = jnp.concatenate(
        [jnp.zeros_like(c0), c0, jnp.maximum(c0, c1),
         jnp.maximum(jnp.maximum(c0, c1), c2)], axis=1)
    onehot = jnp.logical_and(eqf > 0.0, prior == 0.0)
    psel = jnp.where(onehot, probs, 0.0)
    p_ref[...] = psel

    @pl.when(i == 0)
    def _():
        cnt_ref[...] = jnp.zeros_like(cnt_ref)
        psum_ref[...] = jnp.zeros_like(psum_ref)

    cnt_ref[...] += jnp.sum(onehot.astype(jnp.float32), axis=0, keepdims=True)
    psum_ref[...] += jnp.sum(psel, axis=0, keepdims=True)


def _expert_kernel(zg_ref, zi_ref, p_ref, ap_ref,
                   e0w1_ref, e0b1_ref, e0w2_ref, e0b2_ref,
                   e1w1_ref, e1b1_ref, e1w2_ref, e1b2_ref,
                   wq_ref, bq_ref, wk_ref, bk_ref, wv_ref, bv_ref,
                   e2ow_ref, e2ob_ref, e2fw_ref, e2fb_ref,
                   e3aw_ref, e3ab_ref, e3ow_ref, e3ob_ref,
                   out_ref):
    zg = zg_ref[...]
    zi = zi_ref[...]
    zgb = zg.astype(jnp.bfloat16)
    zib = zi.astype(jnp.bfloat16)
    x = jnp.concatenate([zgb, zib], axis=1)

    h0 = jax.nn.relu(_mm(x, e0w1_ref[...], e0b1_ref[...])).astype(jnp.bfloat16)
    out0 = _mm(h0, e0w2_ref[...], e0b2_ref[...])

    prod = (zg * zi).astype(jnp.bfloat16)
    h1 = jax.nn.relu(_mm(prod, e1w1_ref[...], e1b1_ref[...])).astype(jnp.bfloat16)
    out1 = _mm(h1, e1w2_ref[...], e1b2_ref[...])

    q0 = _mm(zgb, wq_ref[...], bq_ref[...])
    q1 = _mm(zib, wq_ref[...], bq_ref[...])
    k0 = _mm(zgb, wk_ref[...], bk_ref[...])
    k1 = _mm(zib, wk_ref[...], bk_ref[...])
    v0 = _mm(zgb, wv_ref[...], bv_ref[...])
    v1 = _mm(zib, wv_ref[...], bv_ref[...])
    scale = 1.0 / (HD ** 0.5)
    ctx_parts = []
    for h in range(NH):
        sl = slice(h * HD, (h + 1) * HD)
        q0h, q1h = q0[:, sl], q1[:, sl]
        k0h, k1h = k0[:, sl], k1[:, sl]
        v0h, v1h = v0[:, sl], v1[:, sl]
        s00 = jnp.sum(q0h * k0h, axis=1, keepdims=True) * scale
        s01 = jnp.sum(q0h * k1h, axis=1, keepdims=True) * scale
        s10 = jnp.sum(q1h * k0h, axis=1, keepdims=True) * scale
        s11 = jnp.sum(q1h * k1h, axis=1, keepdims=True) * scale
        m0 = jnp.maximum(s00, s01)
        a00 = jnp.exp(s00 - m0)
        a01 = jnp.exp(s01 - m0)
        m1 = jnp.maximum(s10, s11)
        a10 = jnp.exp(s10 - m1)
        a11 = jnp.exp(s11 - m1)
        ctx0 = (a00 * v0h + a01 * v1h) / (a00 + a01)
        ctx1 = (a10 * v0h + a11 * v1h) / (a10 + a11)
        ctx_parts.append(0.5 * (ctx0 + ctx1))
    mean_ctx = jnp.concatenate(ctx_parts, axis=1).astype(jnp.bfloat16)
    fused2 = _mm(mean_ctx, e2ow_ref[...], e2ob_ref[...]).astype(jnp.bfloat16)
    out2 = _mm(fused2, e2fw_ref[...], e2fb_ref[...])

    alpha = jax.nn.sigmoid(_mm(x, e3aw_ref[...], e3ab_ref[...]))
    h3 = (alpha * zg + (1.0 - alpha) * zi).astype(jnp.bfloat16)
    out3 = _mm(h3, e3ow_ref[...], e3ob_ref[...])

    p = p_ref[...]
    ap = ap_ref[...]
    acc = jnp.zeros_like(out0)
    for e, oe in enumerate((out0, out1, out2, out3)):
        sel = (p[:, e:e + 1] > 0.0).astype(jnp.float32) * ap[0:1, e:e + 1]
        acc = acc + sel * oe
    out_ref[...] = acc


@functools.partial(jax.jit, static_argnames=())
def kernel(z_gat, z_gin, gate_W, gate_b,
           e0_fc1_W, e0_fc1_b, e0_fc2_W, e0_fc2_b,
           e1_fc1_W, e1_fc1_b, e1_fc2_W, e1_fc2_b,
           e2_in_W, e2_in_b, e2_out_W, e2_out_b, e2_fc_W, e2_fc_b,
           e3_alpha_W, e3_alpha_b, e3_out_W, e3_out_b):
    f32 = jnp.float32
    bf16 = jnp.bfloat16

    grid_g = B // TG
    p_sel, cnt, psum = pl.pallas_call(
        _gate_kernel,
        grid=(grid_g,),
        in_specs=[
            pl.BlockSpec((TG, D), lambda i: (i, 0)),
            pl.BlockSpec((TG, D), lambda i: (i, 0)),
            pl.BlockSpec((4, 2 * D), lambda i: (0, 0)),
            pl.BlockSpec((1, 4), lambda i: (0, 0)),
        ],
        out_specs=[
            pl.BlockSpec((TG, 4), lambda i: (i, 0)),
            pl.BlockSpec((1, 4), lambda i: (0, 0)),
            pl.BlockSpec((1, 4), lambda i: (0, 0)),
        ],
        out_shape=[
            jax.ShapeDtypeStruct((B, 4), f32),
            jax.ShapeDtypeStruct((1, 4), f32),
            jax.ShapeDtypeStruct((1, 4), f32),
        ],
    )(z_gat, z_gin, gate_W, gate_b.reshape(1, 4))

    counts = cnt[0]
    avg_prob = jnp.where(counts > 0, psum[0] / jnp.maximum(counts, 1.0), 0.0)
    aux_loss = jnp.sum((counts / float(B)) ** 2) * 4.0

    wq, wk, wv = jnp.split(e2_in_W, 3, axis=0)
    bq, bk, bv = jnp.split(e2_in_b, 3, axis=0)

    def wcast(w):
        return w.astype(bf16)

    def b2d(b):
        return b.reshape(1, -1).astype(f32)

    weight_args = (
        wcast(e0_fc1_W), b2d(e0_fc1_b), wcast(e0_fc2_W), b2d(e0_fc2_b),
        wcast(e1_fc1_W), b2d(e1_fc1_b), wcast(e1_fc2_W), b2d(e1_fc2_b),
        wcast(wq), b2d(bq), wcast(wk), b2d(bk), wcast(wv), b2d(bv),
        wcast(e2_out_W), b2d(e2_out_b), wcast(e2_fc_W), b2d(e2_fc_b),
        wcast(e3_alpha_W), b2d(e3_alpha_b), wcast(e3_out_W), b2d(e3_out_b),
    )

    def wspec(w):
        return pl.BlockSpec(w.shape, lambda i: tuple(0 for _ in w.shape))

    grid_e = B // TE
    output = pl.pallas_call(
        _expert_kernel,
        grid=(grid_e,),
        in_specs=[
            pl.BlockSpec((TE, D), lambda i: (i, 0)),
            pl.BlockSpec((TE, D), lambda i: (i, 0)),
            pl.BlockSpec((TE, 4), lambda i: (i, 0)),
            pl.BlockSpec((1, 4), lambda i: (0, 0)),
        ] + [wspec(w) for w in weight_args],
        out_specs=pl.BlockSpec((TE, D), lambda i: (i, 0)),
        out_shape=jax.ShapeDtypeStruct((B, D), f32),
    )(z_gat, z_gin, p_sel, avg_prob.reshape(1, 4), *weight_args)

    return output, aux_loss
